# baseline (speedup 1.0000x reference)
import functools

import jax
import jax.numpy as jnp
import numpy as np
from jax import lax
from jax.experimental import pallas as pl
from jax.experimental.pallas import tpu as pltpu
from jax.experimental.pallas import tpu_sc as plsc

_IOTA16 = np.arange(16, dtype=np.int32)

_NC = 2
_NS = 16
_NW = _NC * _NS

_GROW = 100
_RPG = 4


def _ln_t_body(tab_ref, g_ref, b_ref, out_ref):
    x = tab_ref[...]
    mean = jnp.mean(x, axis=0, keepdims=True)
    c = x - mean
    var = jnp.mean(c * c, axis=0, keepdims=True)
    xn = c * lax.rsqrt(var + 1e-5) * g_ref[...] + b_ref[...]
    bn = xn.shape[1]
    q = bn // 4
    xnt = xn.T
    out_ref[...] = jnp.concatenate(
        [xnt[k * q:(k + 1) * q, :] for k in range(4)], axis=1
    )


def _normalize_table_t(table_t, gamma, beta):
    d, v = table_t.shape
    blk = 16384
    return pl.pallas_call(
        _ln_t_body,
        grid=(pl.cdiv(v, blk),),
        in_specs=[
            pl.BlockSpec((d, blk), lambda i: (0, i)),
            pl.BlockSpec((d, 1), lambda i: (0, 0)),
            pl.BlockSpec((d, 1), lambda i: (0, 0)),
        ],
        out_specs=pl.BlockSpec((blk // 4, 128), lambda i: (i, 0)),
        out_shape=jax.ShapeDtypeStruct(
            (pl.cdiv(v, blk) * (blk // 4), 128), jnp.float32
        ),
    )(table_t, gamma.reshape(d, 1), beta.reshape(d, 1))


def _sc_gather(tab, idx_t, batch, hist):
    d = tab.shape[1]
    bt_per_w = batch // 128 // _NW
    n_pairs = hist // 2

    mesh = plsc.VectorSubcoreMesh(core_axis_name="c", subcore_axis_name="s")

    @functools.partial(
        pl.kernel,
        mesh=mesh,
        out_type=jax.ShapeDtypeStruct((hist, d // 8, batch // 128, 8, 128),
                                      jnp.float32),
        compiler_params=pltpu.CompilerParams(
            use_tc_tiling_on_sc=False, needs_layout_passes=False
        ),
        scratch_types=[
            pltpu.VMEM((hist, 128), jnp.int32),
            pltpu.VMEM((2, 128, d), jnp.float32),
            pltpu.VMEM((2, d // 8, 8, 128), jnp.float32),
            pltpu.SemaphoreType.DMA,
            pltpu.SemaphoreType.DMA,
        ],
    )
    def k(tab_hbm, idx_hbm, out_hbm, idx_v, rows_v, rt_v, sem0, sem1):
        wid = lax.axis_index("s") * _NC + lax.axis_index("c")
        sems = (sem0, sem1)
        iota = lax.iota(jnp.int32, 16)

        def gather(b, h):
            return pltpu.make_async_copy(
                tab_hbm.at[idx_v.at[h]], rows_v.at[b], sems[b]
            )

        rots = [(iota + j) & 15 for j in range(16)]
        cidxs = [iota + e0 for e0 in range(0, d, 16)]
        ets = [(iota + e0) // 8 for e0 in range(0, d, 16)]
        eis = [(iota + e0) & 7 for e0 in range(0, d, 16)]

        def transpose(b):
            @plsc.parallel_loop(0, 8, unroll=2)
            def tile_col(ci):
                c0 = ci * 16
                for en in range(d // 16):
                    for j in range(16):
                        ridx = rots[j] + c0
                        val = plsc.load_gather(rows_v.at[b], [ridx, cidxs[en]])
                        plsc.store_scatter(
                            rt_v.at[b], [ets[en], eis[en], ridx], val
                        )

        def store(b, h, btg):
            pltpu.sync_copy(rt_v.at[b], out_hbm.at[h, pl.ds(0, d // 8), btg])

        def bt_body(btl, carry0):
            btg = wid * bt_per_w + btl
            pltpu.sync_copy(idx_hbm.at[:, pl.ds(btg * 128, 128)], idx_v)
            gather(0, 0).start()

            def pair(i, carry):
                h_a = 2 * i
                h_b = h_a + 1
                gather(1, h_b).start()
                gather(0, h_a).wait()
                transpose(0)
                store(0, h_a, btg)

                @pl.when(i + 1 < n_pairs)
                def _():
                    gather(0, h_a + 2).start()

                gather(1, h_b).wait()
                transpose(1)
                store(1, h_b, btg)
                return carry

            lax.fori_loop(0, n_pairs, pair, None)
            return carry0

        lax.fori_loop(0, bt_per_w, bt_body, None)

    return k(tab, idx_t)


def kernel(indices, table, gamma, beta):
    batch, hist = indices.shape
    d = table.shape[1]
    norm_packed = _normalize_table_t(table.T, gamma, beta)
    norm_rows = norm_packed.reshape(norm_packed.shape[0] * 128 // d, d)
    idx = indices.astype(jnp.int32)
    idx = (idx & ~16383) | ((idx & 4095) << 2) | ((idx >> 12) & 3)
    out5 = _sc_gather(norm_rows, idx.T, batch, hist)
    return out5.transpose(2, 4, 0, 1, 3).reshape(batch, hist, d)


# speedup vs baseline: 1.6004x; 1.6004x over previous
import functools

import jax
import jax.numpy as jnp
import numpy as np
from jax import lax
from jax.experimental import pallas as pl
from jax.experimental.pallas import tpu as pltpu
from jax.experimental.pallas import tpu_sc as plsc

_IOTA16 = np.arange(16, dtype=np.int32)

_NC = 2
_NS = 16
_NW = _NC * _NS

_GROW = 100
_RPG = 4


def _ln_t_body(tab_ref, g_ref, b_ref, out_ref):
    x = tab_ref[...]
    mean = jnp.mean(x, axis=0, keepdims=True)
    c = x - mean
    var = jnp.mean(c * c, axis=0, keepdims=True)
    xn = c * lax.rsqrt(var + 1e-5) * g_ref[...] + b_ref[...]
    bn = xn.shape[1]
    q = bn // 4
    xnt = xn.T
    out_ref[...] = jnp.concatenate(
        [xnt[k * q:(k + 1) * q, :] for k in range(4)], axis=1
    )


def _normalize_table_t(table_t, gamma, beta):
    d, v = table_t.shape
    blk = 16384
    return pl.pallas_call(
        _ln_t_body,
        grid=(pl.cdiv(v, blk),),
        in_specs=[
            pl.BlockSpec((d, blk), lambda i: (0, i)),
            pl.BlockSpec((d, 1), lambda i: (0, 0)),
            pl.BlockSpec((d, 1), lambda i: (0, 0)),
        ],
        out_specs=pl.BlockSpec((blk // 4, 128), lambda i: (i, 0)),
        out_shape=jax.ShapeDtypeStruct(
            (pl.cdiv(v, blk) * (blk // 4), 128), jnp.float32
        ),
    )(table_t, gamma.reshape(d, 1), beta.reshape(d, 1))


def _sc_gather(tab, idx_t, batch, hist):
    d = tab.shape[1]
    bt_per_w = batch // 128 // _NW
    n_pairs = hist // 2

    mesh = plsc.VectorSubcoreMesh(core_axis_name="c", subcore_axis_name="s")

    @functools.partial(
        pl.kernel,
        mesh=mesh,
        out_type=jax.ShapeDtypeStruct((hist, d // 8, batch // 128, 8, 128),
                                      jnp.float32),
        compiler_params=pltpu.CompilerParams(
            use_tc_tiling_on_sc=False, needs_layout_passes=False
        ),
        scratch_types=[
            pltpu.VMEM((hist, 128), jnp.int32),
            pltpu.VMEM((2, 128, d), jnp.float32),
            pltpu.VMEM((2, d // 8, 8, 128), jnp.float32),
            pltpu.SemaphoreType.DMA,
            pltpu.SemaphoreType.DMA,
            pltpu.SemaphoreType.DMA,
            pltpu.SemaphoreType.DMA,
        ],
    )
    def k(tab_hbm, idx_hbm, out_hbm, idx_v, rows_v, rt_v, sem0, sem1, osem0, osem1):
        wid = lax.axis_index("s") * _NC + lax.axis_index("c")
        sems = (sem0, sem1)
        iota = lax.iota(jnp.int32, 16)

        def gather(b, h):
            return pltpu.make_async_copy(
                tab_hbm.at[idx_v.at[h]], rows_v.at[b], sems[b]
            )

        rots = [(iota + j) & 15 for j in range(16)]
        cidxs = [iota + e0 for e0 in range(0, d, 16)]
        ets = [(iota + e0) // 8 for e0 in range(0, d, 16)]
        eis = [(iota + e0) & 7 for e0 in range(0, d, 16)]

        def transpose(b):
            def tile_col(ci, carry):
                c0 = ci * 16
                for en in range(d // 16):
                    for j in range(16):
                        ridx = rots[j] + c0
                        val = plsc.load_gather(rows_v.at[b], [ridx, cidxs[en]])
                        plsc.store_scatter(
                            rt_v.at[b], [ets[en], eis[en], ridx], val
                        )
                return carry

            lax.fori_loop(0, 8, tile_col, None)

        osems = (osem0, osem1)

        def store(b, h, btg):
            return pltpu.make_async_copy(
                rt_v.at[b], out_hbm.at[h, pl.ds(0, d // 8), btg], osems[b]
            )

        def bt_body(btl, carry0):
            btg = wid * bt_per_w + btl
            pltpu.sync_copy(idx_hbm.at[:, pl.ds(btg * 128, 128)], idx_v)
            gather(0, 0).start()

            def pair(i, carry):
                h_a = 2 * i
                h_b = h_a + 1
                gather(1, h_b).start()
                gather(0, h_a).wait()

                @pl.when(i > 0)
                def _():
                    store(0, h_a, btg).wait()

                transpose(0)
                store(0, h_a, btg).start()

                @pl.when(i + 1 < n_pairs)
                def _():
                    gather(0, h_a + 2).start()

                gather(1, h_b).wait()

                @pl.when(i > 0)
                def _():
                    store(1, h_b, btg).wait()

                transpose(1)
                store(1, h_b, btg).start()
                return carry

            lax.fori_loop(0, n_pairs, pair, None)
            store(0, 2 * n_pairs - 2, btg).wait()
            store(1, 2 * n_pairs - 1, btg).wait()
            return carry0

        lax.fori_loop(0, bt_per_w, bt_body, None)

    return k(tab, idx_t)


def kernel(indices, table, gamma, beta):
    batch, hist = indices.shape
    d = table.shape[1]
    norm_packed = _normalize_table_t(table.T, gamma, beta)
    norm_rows = norm_packed.reshape(norm_packed.shape[0] * 128 // d, d)
    idx = indices.astype(jnp.int32)
    idx = (idx & ~16383) | ((idx & 4095) << 2) | ((idx >> 12) & 3)
    out5 = _sc_gather(norm_rows, idx.T, batch, hist)
    return out5.transpose(2, 4, 0, 1, 3).reshape(batch, hist, d)
